# Initial kernel scaffold; baseline (speedup 1.0000x reference)
import jax
import jax.numpy as jnp
from jax.experimental import pallas as pl


def kernel(word, pos1, pos2, word_emb, pos1_emb, pos2_emb):
    raise NotImplementedError("write your pallas kernel here")


# speedup vs baseline: 3.3798x; 3.3798x over previous
import functools

import jax
import jax.numpy as jnp
from jax import lax
from jax.experimental import pallas as pl
from jax.experimental.pallas import tpu as pltpu
from jax.experimental.pallas import tpu_sc as plsc

CHUNK = 512
SUB = 128
LANES = 16


def _pad8(n):
    return (n + 7) // 8 * 8


@functools.lru_cache(maxsize=None)
def _build(N, VW, DWP, DW, VP, DP, n_workers):
    per_w = N // n_workers
    n_chunks = per_w // CHUNK
    n_sub = CHUNK // SUB
    DOUT = DW + 2 * DP
    NCH = N // CHUNK
    mesh = plsc.VectorSubcoreMesh(core_axis_name="c", subcore_axis_name="s")

    @functools.partial(
        pl.kernel,
        out_type=jax.ShapeDtypeStruct((DOUT, NCH, CHUNK), jnp.float32),
        mesh=mesh,
        scratch_types=[
            [pltpu.VMEM((SUB,), jnp.int32) for _ in range(n_sub)],
            pltpu.VMEM((CHUNK,), jnp.int32),
            pltpu.VMEM((CHUNK,), jnp.int32),
            pltpu.VMEM((_pad8(VP * DP),), jnp.float32),
            pltpu.VMEM((_pad8(VP * DP),), jnp.float32),
            [pltpu.VMEM((SUB, DWP), jnp.float32) for _ in range(n_sub)],
            pltpu.VMEM((DOUT, 1, CHUNK), jnp.float32),
            pltpu.SemaphoreType.DMA,
        ],
        compiler_params=pltpu.CompilerParams(
            needs_layout_passes=False, use_tc_tiling_on_sc=False),
    )
    def k(word_hbm, pos1_hbm, pos2_hbm, wtab_hbm, p1tab_hbm, p2tab_hbm,
          out_hbm, widx, idx1, idx2, p1tv, p2tv, wrows, outb, sem):
        wid = lax.axis_index("s") * 2 + lax.axis_index("c")
        pltpu.sync_copy(p1tab_hbm, p1tv)
        pltpu.sync_copy(p2tab_hbm, p2tv)
        iota = lax.iota(jnp.int32, LANES)

        def chunk_body(kk, carry):
            ci = wid * n_chunks + kk
            for j in range(n_sub):
                pltpu.sync_copy(word_hbm.at[ci, j], widx[j])
            pltpu.sync_copy(pos1_hbm.at[ci], idx1)
            pltpu.sync_copy(pos2_hbm.at[ci], idx2)
            handles = [
                pltpu.async_copy(wtab_hbm.at[widx[j]], wrows[j], sem)
                for j in range(n_sub)
            ]
            for h in handles:
                h.wait()

            for j in range(n_sub):
                def asm_body(g, carry2, j=j):
                    p16 = g * LANES + iota
                    q = j * SUB + g * LANES
                    r1 = idx1[pl.ds(q, LANES)] * DP
                    r2 = idx2[pl.ds(q, LANES)] * DP
                    for f in range(DW):
                        col = jnp.full((LANES,), f, jnp.int32)
                        outb[f, 0, pl.ds(q, LANES)] = (
                            plsc.load_gather(wrows[j], [p16, col]))
                    for c in range(DP):
                        outb[DW + c, 0, pl.ds(q, LANES)] = (
                            plsc.load_gather(p1tv, [r1 + c]))
                        outb[DW + DP + c, 0, pl.ds(q, LANES)] = (
                            plsc.load_gather(p2tv, [r2 + c]))
                    return carry2
                lax.fori_loop(0, SUB // LANES, asm_body, 0)

            pltpu.sync_copy(outb, out_hbm.at[:, pl.ds(ci, 1), :])
            return carry

        lax.fori_loop(0, n_chunks, chunk_body, 0)

    return k


def kernel(word, pos1, pos2, word_emb, pos1_emb, pos2_emb):
    B, L = word.shape
    N = B * L
    VW, DW = word_emb.shape
    VP, DP = pos1_emb.shape
    DOUT = DW + 2 * DP
    NCH = N // CHUNK

    def blockify(a):
        return (a.reshape(B // 128, 128, L // 8, 8)
                 .transpose(2, 0, 3, 1)
                 .reshape(N).astype(jnp.int32))

    DWP = _pad8(DW)
    k = _build(N, VW, DWP, DW, VP, DP, 32)
    pad = _pad8(VP * DP) - VP * DP
    out = k(
        blockify(word).reshape(NCH, CHUNK // SUB, SUB),
        blockify(pos1).reshape(NCH, CHUNK),
        blockify(pos2).reshape(NCH, CHUNK),
        jnp.pad(word_emb, ((0, 0), (0, DWP - DW))),
        jnp.pad(pos1_emb.reshape(-1), (0, pad)),
        jnp.pad(pos2_emb.reshape(-1), (0, pad)),
    )
    out = (out.reshape(DOUT, L // 8, B // 128, 8, 128)
              .transpose(2, 4, 1, 3, 0)
              .reshape(B, L, DOUT))
    return out


# speedup vs baseline: 3.5032x; 1.0365x over previous
import functools

import jax
import jax.numpy as jnp
from jax import lax
from jax.experimental import pallas as pl
from jax.experimental.pallas import tpu as pltpu
from jax.experimental.pallas import tpu_sc as plsc

CHUNK = 512
SUB = 128
LANES = 16


def _pad8(n):
    return (n + 7) // 8 * 8


@functools.lru_cache(maxsize=None)
def _build(N, VW, DWP, DW, VP, DP, n_workers):
    per_w = N // n_workers
    n_chunks = per_w // CHUNK
    n_sub = CHUNK // SUB
    DOUT = DW + 2 * DP
    NCH = N // CHUNK
    mesh = plsc.VectorSubcoreMesh(core_axis_name="c", subcore_axis_name="s")

    @functools.partial(
        pl.kernel,
        out_type=jax.ShapeDtypeStruct((DOUT, NCH, CHUNK), jnp.float32),
        mesh=mesh,
        scratch_types=[
            [[pltpu.VMEM((SUB,), jnp.int32) for _ in range(n_sub)]
             for _ in range(2)],
            [pltpu.VMEM((CHUNK,), jnp.int32) for _ in range(2)],
            [pltpu.VMEM((CHUNK,), jnp.int32) for _ in range(2)],
            pltpu.VMEM((_pad8(VP * DP),), jnp.float32),
            pltpu.VMEM((_pad8(VP * DP),), jnp.float32),
            [[pltpu.VMEM((SUB, DWP), jnp.float32) for _ in range(n_sub)]
             for _ in range(2)],
            pltpu.VMEM((DOUT, 1, CHUNK), jnp.float32),
            [pltpu.SemaphoreType.DMA for _ in range(2)],
        ],
        compiler_params=pltpu.CompilerParams(
            needs_layout_passes=False, use_tc_tiling_on_sc=False),
    )
    def k(word_hbm, pos1_hbm, pos2_hbm, wtab_hbm, p1tab_hbm, p2tab_hbm,
          out_hbm, widx, idx1, idx2, p1tv, p2tv, wrows, outb, sem):
        wid = lax.axis_index("s") * 2 + lax.axis_index("c")
        pltpu.sync_copy(p1tab_hbm, p1tv)
        pltpu.sync_copy(p2tab_hbm, p2tv)
        iota = lax.iota(jnp.int32, LANES)
        cbase = wid * n_chunks

        def load_and_fire(pb, ci):
            for j in range(n_sub):
                pltpu.sync_copy(word_hbm.at[ci, j], widx[pb][j])
            pltpu.sync_copy(pos1_hbm.at[ci], idx1[pb])
            pltpu.sync_copy(pos2_hbm.at[ci], idx2[pb])
            for j in range(n_sub):
                pltpu.async_copy(wtab_hbm.at[widx[pb][j]], wrows[pb][j],
                                 sem[pb])

        def drain(pb):
            for j in range(n_sub):
                pltpu.make_async_copy(wtab_hbm.at[widx[pb][j]], wrows[pb][j],
                                      sem[pb]).wait()

        def consume(pb, ci):
            drain(pb)
            for j in range(n_sub):
                def asm_body(g, carry2, j=j):
                    p16 = g * LANES + iota
                    q = j * SUB + g * LANES
                    r1 = idx1[pb][pl.ds(q, LANES)] * DP
                    r2 = idx2[pb][pl.ds(q, LANES)] * DP
                    for f in range(DW):
                        col = jnp.full((LANES,), f, jnp.int32)
                        outb[f, 0, pl.ds(q, LANES)] = (
                            plsc.load_gather(wrows[pb][j], [p16, col]))
                    for c in range(DP):
                        outb[DW + c, 0, pl.ds(q, LANES)] = (
                            plsc.load_gather(p1tv, [r1 + c]))
                        outb[DW + DP + c, 0, pl.ds(q, LANES)] = (
                            plsc.load_gather(p2tv, [r2 + c]))
                    return carry2
                lax.fori_loop(0, SUB // LANES, asm_body, 0)
            pltpu.sync_copy(outb, out_hbm.at[:, pl.ds(ci, 1), :])

        load_and_fire(0, cbase)

        def pair_body(kk, carry):
            c0 = cbase + 2 * kk
            load_and_fire(1, c0 + 1)
            consume(0, c0)
            load_and_fire(0, jnp.minimum(c0 + 2, NCH - 1))
            consume(1, c0 + 1)
            return carry

        lax.fori_loop(0, n_chunks // 2, pair_body, 0)
        drain(0)

    return k


def kernel(word, pos1, pos2, word_emb, pos1_emb, pos2_emb):
    B, L = word.shape
    N = B * L
    VW, DW = word_emb.shape
    VP, DP = pos1_emb.shape
    DOUT = DW + 2 * DP
    NCH = N // CHUNK

    def blockify(a):
        return (a.reshape(B // 128, 128, L // 8, 8)
                 .transpose(2, 0, 3, 1)
                 .reshape(N).astype(jnp.int32))

    DWP = _pad8(DW)
    k = _build(N, VW, DWP, DW, VP, DP, 32)
    pad = _pad8(VP * DP) - VP * DP
    out = k(
        blockify(word).reshape(NCH, CHUNK // SUB, SUB),
        blockify(pos1).reshape(NCH, CHUNK),
        blockify(pos2).reshape(NCH, CHUNK),
        jnp.pad(word_emb, ((0, 0), (0, DWP - DW))),
        jnp.pad(pos1_emb.reshape(-1), (0, pad)),
        jnp.pad(pos2_emb.reshape(-1), (0, pad)),
    )
    out = (out.reshape(DOUT, L // 8, B // 128, 8, 128)
              .transpose(2, 4, 1, 3, 0)
              .reshape(B, L, DOUT))
    return out


# speedup vs baseline: 3.7685x; 1.0757x over previous
import functools

import jax
import jax.numpy as jnp
from jax import lax
from jax.experimental import pallas as pl
from jax.experimental.pallas import tpu as pltpu
from jax.experimental.pallas import tpu_sc as plsc

CHUNK = 512
SUB = 128
LANES = 16


def _pad8(n):
    return (n + 7) // 8 * 8


@functools.lru_cache(maxsize=None)
def _build(N, VW, DWP, DW, VP, DP, n_workers):
    per_w = N // n_workers
    n_chunks = per_w // CHUNK
    n_sub = CHUNK // SUB
    DOUT = DW + 2 * DP
    NCH = N // CHUNK
    mesh = plsc.VectorSubcoreMesh(core_axis_name="c", subcore_axis_name="s")

    @functools.partial(
        pl.kernel,
        out_type=jax.ShapeDtypeStruct((DOUT, NCH, CHUNK), jnp.float32),
        mesh=mesh,
        scratch_types=[
            [pltpu.VMEM((3 * n_sub, SUB), jnp.int32)
             for _ in range(2)],
            pltpu.VMEM((_pad8(VP * DP),), jnp.float32),
            pltpu.VMEM((_pad8(VP * DP),), jnp.float32),
            [[pltpu.VMEM((SUB, DWP), jnp.float32) for _ in range(n_sub)]
             for _ in range(2)],
            [pltpu.VMEM((DOUT, 1, CHUNK), jnp.float32)
             for _ in range(2)],
            [pltpu.SemaphoreType.DMA for _ in range(4)],
        ],
        compiler_params=pltpu.CompilerParams(
            needs_layout_passes=False, use_tc_tiling_on_sc=False),
    )
    def k(idx_hbm, wtab_hbm, p1tab_hbm, p2tab_hbm,
          out_hbm, idxall, p1tv, p2tv, wrows, outb, sem):
        wid = lax.axis_index("s") * 2 + lax.axis_index("c")
        pltpu.sync_copy(p1tab_hbm, p1tv)
        pltpu.sync_copy(p2tab_hbm, p2tv)
        iota = lax.iota(jnp.int32, LANES)
        cbase = wid * n_chunks

        def load_and_fire(pb, ci):
            pltpu.sync_copy(idx_hbm.at[ci], idxall[pb])
            for j in range(n_sub):
                pltpu.async_copy(wtab_hbm.at[idxall[pb].at[j]], wrows[pb][j],
                                 sem[pb])

        def drain(pb):
            for j in range(n_sub):
                pltpu.make_async_copy(wtab_hbm.at[idxall[pb].at[j]],
                                      wrows[pb][j], sem[pb]).wait()

        def fire_write(pb, ci):
            pltpu.async_copy(outb[pb], out_hbm.at[:, pl.ds(ci, 1), :],
                             sem[2 + pb])

        def drain_write(pb, ci):
            pltpu.make_async_copy(outb[pb], out_hbm.at[:, pl.ds(ci, 1), :],
                                  sem[2 + pb]).wait()

        def consume(pb, ci):
            drain(pb)
            drain_write(pb, ci)
            for j in range(n_sub):
                def asm_body(g, carry2, j=j):
                    p16 = g * LANES + iota
                    q = j * SUB + g * LANES
                    r1 = idxall[pb][n_sub + j, pl.ds(g * LANES, LANES)] * DP
                    r2 = idxall[pb][2 * n_sub + j,
                                    pl.ds(g * LANES, LANES)] * DP
                    for f in range(DW):
                        col = jnp.full((LANES,), f, jnp.int32)
                        outb[pb][f, 0, pl.ds(q, LANES)] = (
                            plsc.load_gather(wrows[pb][j], [p16, col]))
                    for c in range(DP):
                        outb[pb][DW + c, 0, pl.ds(q, LANES)] = (
                            plsc.load_gather(p1tv, [r1 + c]))
                        outb[pb][DW + DP + c, 0, pl.ds(q, LANES)] = (
                            plsc.load_gather(p2tv, [r2 + c]))
                    return carry2
                lax.fori_loop(0, SUB // LANES, asm_body, 0)
            fire_write(pb, ci)

        load_and_fire(0, cbase)
        fire_write(0, cbase)
        fire_write(1, cbase + 1)

        def pair_body(kk, carry):
            c0 = cbase + 2 * kk
            load_and_fire(1, c0 + 1)
            consume(0, c0)
            load_and_fire(0, jnp.minimum(c0 + 2, NCH - 1))
            consume(1, c0 + 1)
            return carry

        lax.fori_loop(0, n_chunks // 2, pair_body, 0)
        drain(0)
        drain_write(0, cbase)
        drain_write(1, cbase + 1)

    return k


def kernel(word, pos1, pos2, word_emb, pos1_emb, pos2_emb):
    B, L = word.shape
    N = B * L
    VW, DW = word_emb.shape
    VP, DP = pos1_emb.shape
    DOUT = DW + 2 * DP
    NCH = N // CHUNK

    def blockify(a):
        return (a.reshape(B // 128, 128, L // 8, 8)
                 .transpose(2, 0, 3, 1)
                 .reshape(N).astype(jnp.int32))

    DWP = _pad8(DW)
    k = _build(N, VW, DWP, DW, VP, DP, 32)
    pad = _pad8(VP * DP) - VP * DP
    n_sub = CHUNK // SUB
    idxcat = jnp.concatenate(
        [blockify(word).reshape(NCH, n_sub, SUB),
         blockify(pos1).reshape(NCH, n_sub, SUB),
         blockify(pos2).reshape(NCH, n_sub, SUB)], axis=1)
    out = k(
        idxcat,
        jnp.pad(word_emb, ((0, 0), (0, DWP - DW))),
        jnp.pad(pos1_emb.reshape(-1), (0, pad)),
        jnp.pad(pos2_emb.reshape(-1), (0, pad)),
    )
    out = (out.reshape(DOUT, L // 8, B // 128, 8, 128)
              .transpose(2, 4, 1, 3, 0)
              .reshape(B, L, DOUT))
    return out
